# Initial kernel scaffold; baseline (speedup 1.0000x reference)
import jax
import jax.numpy as jnp
from jax.experimental import pallas as pl


def kernel(track_uri_can, artist_uri_can, album_uri_can, track_name_can, artist_name_can, album_name_can, artist_genres_can, track_key_can, track_mode_can, time_signature_can, duration_ms_can, track_pop_can, artist_pop_can, artists_followers_can, track_danceability_can, track_energy_can, track_loudness_can, track_speechiness_can, track_acousticness_can, track_instrumentalness_can, track_liveness_can, track_valence_can, track_tempo_can, E_track_uri, E_artist_uri, E_album_uri, E_track_name, E_artist_name, E_album_name, E_genres, E_duration, E_track_pop, E_artist_pop, E_followers, E_dance, E_energy, E_key, E_loud, E_mode, E_speech, E_acoustic, E_instr, E_live, E_valence, E_tempo, E_timesig, cross_U, cross_V, cross_b, W1, b1, W2, b2, W3, b3, ln_g, ln_b):
    raise NotImplementedError("write your pallas kernel here")


# speedup vs baseline: 1.1543x; 1.1543x over previous
import functools

import jax
import jax.numpy as jnp
import numpy as np
from jax import lax
from jax.experimental import pallas as pl
from jax.experimental.pallas import tpu as pltpu
from jax.experimental.pallas import tpu_sc as plsc

B = 4096
D = 32
L = 20
NC = 2
NS = 16
NW = NC * NS
BPW = B // NW

BLK = 512


def _sc_body(idx1, idx2, idx3, idxT4, idxT5, idxT6, idxT7,
             tab1, tab2, tab3, tab4, tab5, tab6, tab7,
             out1, out2, out3, out4, out5, out6, out7,
             idx_v, idxT_v, tok_buf, row_buf, sem):
    wid = lax.axis_index("s") * NC + lax.axis_index("c")
    base = wid * BPW

    def direct(idx_hbm, tab_hbm, out_hbm):
        pltpu.sync_copy(idx_hbm.at[pl.ds(base, BPW)], idx_v)
        pltpu.async_copy(tab_hbm.at[idx_v], row_buf, sem).wait()
        pltpu.sync_copy(row_buf, out_hbm.at[pl.ds(base, BPW)])

    def pooled(idxT_hbm, tab_hbm, out_hbm):
        pltpu.sync_copy(idxT_hbm.at[:, pl.ds(base, BPW)], idxT_v)
        handles = [
            pltpu.async_copy(tab_hbm.at[idxT_v.at[t]], tok_buf.at[t], sem)
            for t in range(L)
        ]
        for h in handles:
            h.wait()

        def body(r, carry):
            for half in range(2):
                sl = pl.ds(half * 16, 16)
                acc = tok_buf[0, r, sl]
                for t in range(1, L):
                    acc = acc + tok_buf[t, r, sl]
                row_buf[r, sl] = acc
            return carry

        lax.fori_loop(0, BPW, body, 0)
        pltpu.sync_copy(row_buf, out_hbm.at[pl.ds(base, BPW)])

    direct(idx1, tab1, out1)
    direct(idx2, tab2, out2)
    direct(idx3, tab3, out3)
    pooled(idxT4, tab4, out4)
    pooled(idxT5, tab5, out5)
    pooled(idxT6, tab6, out6)
    pooled(idxT7, tab7, out7)


def _sc_gather(idx1, idx2, idx3, idxT4, idxT5, idxT6, idxT7,
               tab1, tab2, tab3, tab4, tab5, tab6, tab7):
    f = jax.ShapeDtypeStruct((B, D), jnp.float32)
    kern = pl.kernel(
        _sc_body,
        out_type=[f] * 7,
        mesh=plsc.VectorSubcoreMesh(core_axis_name="c", subcore_axis_name="s"),
        compiler_params=pltpu.CompilerParams(use_tc_tiling_on_sc=False),
        scratch_types=[
            pltpu.VMEM((BPW,), jnp.int32),
            pltpu.VMEM((L, BPW), jnp.int32),
            pltpu.VMEM((L, BPW, D), jnp.float32),
            pltpu.VMEM((BPW, D), jnp.float32),
            pltpu.SemaphoreType.DMA,
        ],
    )
    return kern(idx1, idx2, idx3, idxT4, idxT5, idxT6, idxT7,
                tab1, tab2, tab3, tab4, tab5, tab6, tab7)


def _tower_body(sc0, sc1, sc2, sc3, sc4, sc5, sc6,
                vals_ref, idxp_ref, gpad_ref, bins_ref, dtab_ref, ptab_ref,
                g0_ref, U_ref, V_ref, cb_ref, W1_ref, b1_ref, W2_ref, b2_ref,
                W3_ref, b3_ref, lg_ref, lb_ref, out_ref):
    f32 = jnp.float32
    vals = vals_ref[...]
    idxp = idxp_ref[...]
    gpad = gpad_ref[...]
    bins = bins_ref[...]
    iota = lax.broadcasted_iota(jnp.int32, (BLK, 32), 1)

    def onehot_lookup(idx_col, tab):
        oh = (iota == idx_col).astype(f32)
        return jnp.dot(oh, tab, preferred_element_type=f32)

    def disc_piece(f):
        v = vals[:, f:f + 1]
        bb = bins[f:f + 1, :]
        idx_col = jnp.sum((v >= bb).astype(jnp.int32), axis=1, keepdims=True)
        return onehot_lookup(idx_col, dtab_ref[f])

    def tiny_piece(j):
        idx_col = idxp[:, j:j + 1]
        return onehot_lookup(idx_col, ptab_ref[j])

    inv_l = jnp.float32(1.0 / L)
    cnt = jnp.sum((gpad != 0).astype(f32), axis=1, keepdims=True)
    g_num = sc6[...] - (jnp.float32(L) - cnt) * g0_ref[...]
    g_piece = g_num / jnp.maximum(cnt, 1.0)

    pieces = [
        sc0[...],
        sc3[...] * inv_l,
        sc1[...],
        sc4[...] * inv_l,
        sc2[...],
        sc5[...] * inv_l,
        disc_piece(0),
        disc_piece(1),
        disc_piece(2),
        g_piece,
        disc_piece(3),
        disc_piece(4),
        disc_piece(5),
        tiny_piece(0),
        disc_piece(6),
        tiny_piece(1),
        disc_piece(7),
        disc_piece(8),
        disc_piece(9),
        disc_piece(10),
        disc_piece(11),
        disc_piece(12),
        tiny_piece(2),
    ]
    x0 = jnp.concatenate(pieces, axis=1)

    t = jnp.dot(x0, U_ref[...], preferred_element_type=f32)
    v = jnp.dot(t, V_ref[...], preferred_element_type=f32) + cb_ref[...]
    xc = x0 * v + x0

    h = jnp.dot(xc, W1_ref[...], preferred_element_type=f32) + b1_ref[...]
    h = jnp.maximum(h, 0.0)
    h = jnp.dot(h, W2_ref[...], preferred_element_type=f32) + b2_ref[...]
    h = jnp.maximum(h, 0.0)
    h = jnp.dot(h, W3_ref[...], preferred_element_type=f32) + b3_ref[...]

    mu = jnp.mean(h, axis=1, keepdims=True)
    var = jnp.mean((h - mu) ** 2, axis=1, keepdims=True)
    hn = (h - mu) / jnp.sqrt(var + 1e-3)
    out_ref[...] = hn * lg_ref[...] + lb_ref[...]


def _tower(sc_outs, vals_pack, idx_pack, gpad, bins_pack, dtabs, ptabs, g0,
           U, V, cb, W1, b1, W2, b2, W3, b3, lg, lb):
    grid = (B // BLK,)

    def bspec(shape, mapped=False):
        if mapped:
            return pl.BlockSpec((BLK,) + shape[1:],
                                lambda i: (i,) + (0,) * (len(shape) - 1))
        return pl.BlockSpec(shape, lambda i: (0,) * len(shape))

    in_specs = (
        [bspec((B, D), True)] * 7 +
        [bspec(vals_pack.shape, True), bspec(idx_pack.shape, True),
         bspec(gpad.shape, True),
         bspec(bins_pack.shape), bspec(dtabs.shape), bspec(ptabs.shape),
         bspec(g0.shape), bspec(U.shape), bspec(V.shape), bspec(cb.shape),
         bspec(W1.shape), bspec(b1.shape), bspec(W2.shape), bspec(b2.shape),
         bspec(W3.shape), bspec(b3.shape), bspec(lg.shape), bspec(lb.shape)]
    )
    return pl.pallas_call(
        _tower_body,
        grid=grid,
        in_specs=in_specs,
        out_specs=pl.BlockSpec((BLK, 128), lambda i: (i, 0)),
        out_shape=jax.ShapeDtypeStruct((B, 128), jnp.float32),
    )(*sc_outs, vals_pack, idx_pack, gpad, bins_pack, dtabs, ptabs, g0,
      U, V, cb, W1, b1, W2, b2, W3, b3, lg, lb)


def kernel(track_uri_can, artist_uri_can, album_uri_can, track_name_can,
           artist_name_can, album_name_can, artist_genres_can, track_key_can,
           track_mode_can, time_signature_can, duration_ms_can, track_pop_can,
           artist_pop_can, artists_followers_can, track_danceability_can,
           track_energy_can, track_loudness_can, track_speechiness_can,
           track_acousticness_can, track_instrumentalness_can,
           track_liveness_can, track_valence_can, track_tempo_can,
           E_track_uri, E_artist_uri, E_album_uri, E_track_name,
           E_artist_name, E_album_name, E_genres, E_duration, E_track_pop,
           E_artist_pop, E_followers, E_dance, E_energy, E_key, E_loud,
           E_mode, E_speech, E_acoustic, E_instr, E_live, E_valence, E_tempo,
           E_timesig, cross_U, cross_V, cross_b, W1, b1, W2, b2, W3, b3,
           ln_g, ln_b):
    i32 = jnp.int32
    f32 = jnp.float32

    idx1 = track_uri_can.astype(i32)
    idx2 = artist_uri_can.astype(i32)
    idx3 = album_uri_can.astype(i32)
    idxT4 = track_name_can.astype(i32).T
    idxT5 = artist_name_can.astype(i32).T
    idxT6 = album_name_can.astype(i32).T
    idxT7 = artist_genres_can.astype(i32).T

    sc_outs = _sc_gather(idx1, idx2, idx3, idxT4, idxT5, idxT6, idxT7,
                         E_track_uri, E_artist_uri, E_album_uri,
                         E_track_name, E_artist_name, E_album_name, E_genres)

    vals = [duration_ms_can, track_pop_can, artist_pop_can,
            artists_followers_can, track_danceability_can, track_energy_can,
            track_loudness_can, track_speechiness_can, track_acousticness_can,
            track_instrumentalness_can, track_liveness_can, track_valence_can,
            track_tempo_can]
    maxvs = [20744575.0, 100.0, 100.0, 94437255.0, 1.0, 1.0, 5.0, 1.0, 1.0,
             1.0, 1.0, 1.0, 250.0]
    vals_pack = jnp.pad(jnp.stack([v.astype(f32) for v in vals], axis=1),
                        ((0, 0), (0, 3)))

    idx_pack = jnp.pad(jnp.stack([track_key_can.astype(i32),
                                  track_mode_can.astype(i32),
                                  time_signature_can.astype(i32)], axis=1),
                       ((0, 0), (0, 5)))

    gpad = jnp.pad(artist_genres_can.astype(i32), ((0, 0), (0, 32 - L)))

    bins_np = np.full((16, 32), np.inf, dtype=np.float32)
    for f, mv in enumerate(maxvs):
        bins_np[f, :20] = np.linspace(0.0, mv, 20, dtype=np.float32)
    bins_pack = jnp.asarray(bins_np)

    def pad32(tab):
        return jnp.pad(tab.astype(f32), ((0, 32 - tab.shape[0]), (0, 0)))

    dtabs = jnp.stack([pad32(t) for t in
                       [E_duration, E_track_pop, E_artist_pop, E_followers,
                        E_dance, E_energy, E_loud, E_speech, E_acoustic,
                        E_instr, E_live, E_valence, E_tempo]])
    ptabs = jnp.stack([pad32(E_key), pad32(E_mode), pad32(E_timesig)])
    g0 = E_genres[0:1].astype(f32)

    return _tower(sc_outs, vals_pack, idx_pack, gpad, bins_pack, dtabs, ptabs,
                  g0, cross_U.astype(f32), cross_V.astype(f32),
                  cross_b.astype(f32).reshape(1, -1), W1.astype(f32),
                  b1.astype(f32).reshape(1, -1), W2.astype(f32),
                  b2.astype(f32).reshape(1, -1), W3.astype(f32),
                  b3.astype(f32).reshape(1, -1), ln_g.astype(f32).reshape(1, -1),
                  ln_b.astype(f32).reshape(1, -1))


# speedup vs baseline: 1.1598x; 1.0047x over previous
import functools

import jax
import jax.numpy as jnp
import numpy as np
from jax import lax
from jax.experimental import pallas as pl
from jax.experimental.pallas import tpu as pltpu
from jax.experimental.pallas import tpu_sc as plsc

B = 4096
D = 32
L = 20
NC = 2
NS = 16
NW = NC * NS
BPW = B // NW

BLK = 512

N_TRACK = 2249562
N_ARTIST = 295861
N_ALBUM = 734685


def _quad_rows(n):
    return n * D // 128


def _sc_direct_body(idx1, idx2, idx3, qtab1, qtab2, qtab3,
                    out1, out2, out3, idx_v, buf, sem):
    wid = lax.axis_index("s") * NC + lax.axis_index("c")
    base = wid * BPW

    def direct(idx_hbm, qtab_hbm, out_hbm, qmax):
        pltpu.sync_copy(idx_hbm.at[pl.ds(base, BPW)], idx_v)
        for j in range(BPW // 16):
            sl = pl.ds(j * 16, 16)
            idx_v[sl] = jnp.minimum(
                lax.shift_right_logical(idx_v[sl], 2), qmax)
        pltpu.async_copy(qtab_hbm.at[idx_v], buf, sem).wait()
        pltpu.sync_copy(buf, out_hbm.at[pl.ds(base, BPW)])

    direct(idx1, qtab1, out1, _quad_rows(N_TRACK) - 1)
    direct(idx2, qtab2, out2, _quad_rows(N_ARTIST) - 1)
    direct(idx3, qtab3, out3, _quad_rows(N_ALBUM) - 1)


def _sc_direct(idx1, idx2, idx3, qtab1, qtab2, qtab3):
    o = jax.ShapeDtypeStruct((B, 128), jnp.float32)
    kern = pl.kernel(
        _sc_direct_body,
        out_type=[o] * 3,
        mesh=plsc.VectorSubcoreMesh(core_axis_name="c", subcore_axis_name="s"),
        scratch_types=[
            pltpu.VMEM((BPW,), jnp.int32),
            pltpu.VMEM((BPW, 128), jnp.float32),
            pltpu.SemaphoreType.DMA,
        ],
    )
    return kern(idx1, idx2, idx3, qtab1, qtab2, qtab3)


def _sc_pooled_body(idxT4, idxT5, idxT6, idxT7, tab4, tab5, tab6, tab7,
                    out4, out5, out6, out7, idxT_v, tok_buf, row_buf, sem):
    wid = lax.axis_index("s") * NC + lax.axis_index("c")
    base = wid * BPW

    def pooled(idxT_hbm, tab_hbm, out_hbm):
        pltpu.sync_copy(idxT_hbm.at[:, pl.ds(base, BPW)], idxT_v)
        handles = [
            pltpu.async_copy(tab_hbm.at[idxT_v.at[t]], tok_buf.at[t], sem)
            for t in range(L)
        ]
        for h in handles:
            h.wait()

        def body(r, carry):
            for half in range(2):
                sl = pl.ds(half * 16, 16)
                acc = tok_buf[0, r, sl]
                for t in range(1, L):
                    acc = acc + tok_buf[t, r, sl]
                row_buf[r, sl] = acc
            return carry

        lax.fori_loop(0, BPW, body, 0)
        pltpu.sync_copy(row_buf, out_hbm.at[pl.ds(base, BPW)])

    pooled(idxT4, tab4, out4)
    pooled(idxT5, tab5, out5)
    pooled(idxT6, tab6, out6)
    pooled(idxT7, tab7, out7)


def _sc_pooled(idxT4, idxT5, idxT6, idxT7, tab4, tab5, tab6, tab7):
    o = jax.ShapeDtypeStruct((B, D), jnp.float32)
    kern = pl.kernel(
        _sc_pooled_body,
        out_type=[o] * 4,
        mesh=plsc.VectorSubcoreMesh(core_axis_name="c", subcore_axis_name="s"),
        compiler_params=pltpu.CompilerParams(use_tc_tiling_on_sc=False),
        scratch_types=[
            pltpu.VMEM((L, BPW), jnp.int32),
            pltpu.VMEM((L, BPW, D), jnp.float32),
            pltpu.VMEM((BPW, D), jnp.float32),
            pltpu.SemaphoreType.DMA,
        ],
    )
    return kern(idxT4, idxT5, idxT6, idxT7, tab4, tab5, tab6, tab7)


def _tower_body(sc0, sc1, sc2, sc3, sc4, sc5, sc6,
                vals_ref, idxp_ref, gpad_ref, bins_ref, dtab_ref, ptab_ref,
                g0_ref, aux_ref, U_ref, V_ref, cb_ref, W1_ref, b1_ref,
                W2_ref, b2_ref, W3_ref, b3_ref, lg_ref, lb_ref, out_ref):
    f32 = jnp.float32
    vals = vals_ref[...]
    idxp = idxp_ref[...]
    gpad = gpad_ref[...]
    bins = bins_ref[...]
    iota = lax.broadcasted_iota(jnp.int32, (BLK, 32), 1)

    def onehot_lookup(idx_col, tab):
        oh = (iota == idx_col).astype(f32)
        return jnp.dot(oh, tab, preferred_element_type=f32)

    def disc_piece(f):
        v = vals[:, f:f + 1]
        bb = bins[f:f + 1, :]
        idx_col = jnp.sum((v >= bb).astype(jnp.int32), axis=1, keepdims=True)
        return onehot_lookup(idx_col, dtab_ref[f])

    def tiny_piece(j):
        idx_col = idxp[:, j:j + 1]
        return onehot_lookup(idx_col, ptab_ref[j])

    def quad_select(quad_ref, j, aux_start):
        idx_col = idxp[:, 3 + j:4 + j]
        quad = quad_ref[...]
        mod = idx_col & 3
        piece = jnp.where(mod == 0, quad[:, 0:D], 0.0)
        for k in range(1, 4):
            piece = jnp.where(mod == k, quad[:, k * D:(k + 1) * D], piece)
        if aux_start is not None:
            aux_piece = onehot_lookup(idx_col - aux_start, aux_ref[...])
            piece = jnp.where(idx_col >= aux_start, aux_piece, piece)
        return piece

    def tok_sum(ref):
        acc = ref[0]
        for t in range(1, L):
            acc = acc + ref[t]
        return acc

    inv_l = jnp.float32(1.0 / L)
    cnt = jnp.sum((gpad != 0).astype(f32), axis=1, keepdims=True)
    g_num = sc6[...] - (jnp.float32(L) - cnt) * g0_ref[...]
    g_piece = g_num / jnp.maximum(cnt, 1.0)

    pieces = [
        quad_select(sc0, 0, 4 * _quad_rows(N_TRACK)),
        sc3[...] * inv_l,
        quad_select(sc1, 1, None),
        sc4[...] * inv_l,
        quad_select(sc2, 2, None),
        sc5[...] * inv_l,
        disc_piece(0),
        disc_piece(1),
        disc_piece(2),
        g_piece,
        disc_piece(3),
        disc_piece(4),
        disc_piece(5),
        tiny_piece(0),
        disc_piece(6),
        tiny_piece(1),
        disc_piece(7),
        disc_piece(8),
        disc_piece(9),
        disc_piece(10),
        disc_piece(11),
        disc_piece(12),
        tiny_piece(2),
    ]
    x0 = jnp.concatenate(pieces, axis=1)

    t = jnp.dot(x0, U_ref[...], preferred_element_type=f32)
    v = jnp.dot(t, V_ref[...], preferred_element_type=f32) + cb_ref[...]
    xc = x0 * v + x0

    h = jnp.dot(xc, W1_ref[...], preferred_element_type=f32) + b1_ref[...]
    h = jnp.maximum(h, 0.0)
    h = jnp.dot(h, W2_ref[...], preferred_element_type=f32) + b2_ref[...]
    h = jnp.maximum(h, 0.0)
    h = jnp.dot(h, W3_ref[...], preferred_element_type=f32) + b3_ref[...]

    mu = jnp.mean(h, axis=1, keepdims=True)
    var = jnp.mean((h - mu) ** 2, axis=1, keepdims=True)
    hn = (h - mu) / jnp.sqrt(var + 1e-3)
    out_ref[...] = hn * lg_ref[...] + lb_ref[...]


def _tower(scd, scp, vals_pack, idx_pack, gpad, bins_pack, dtabs, ptabs, g0,
           aux, U, V, cb, W1, b1, W2, b2, W3, b3, lg, lb):
    grid = (B // BLK,)

    def bspec(shape, mapped=False):
        if mapped:
            return pl.BlockSpec((BLK,) + shape[1:],
                                lambda i: (i,) + (0,) * (len(shape) - 1))
        return pl.BlockSpec(shape, lambda i: (0,) * len(shape))

    in_specs = (
        [bspec((B, 128), True)] * 3 + [bspec((B, D), True)] * 4 +
        [bspec(vals_pack.shape, True), bspec(idx_pack.shape, True),
         bspec(gpad.shape, True),
         bspec(bins_pack.shape), bspec(dtabs.shape), bspec(ptabs.shape),
         bspec(g0.shape), bspec(aux.shape), bspec(U.shape), bspec(V.shape),
         bspec(cb.shape), bspec(W1.shape), bspec(b1.shape), bspec(W2.shape),
         bspec(b2.shape), bspec(W3.shape), bspec(b3.shape), bspec(lg.shape),
         bspec(lb.shape)]
    )
    return pl.pallas_call(
        _tower_body,
        grid=grid,
        in_specs=in_specs,
        out_specs=pl.BlockSpec((BLK, 128), lambda i: (i, 0)),
        out_shape=jax.ShapeDtypeStruct((B, 128), jnp.float32),
    )(*scd, *scp, vals_pack, idx_pack, gpad, bins_pack, dtabs, ptabs, g0,
      aux, U, V, cb, W1, b1, W2, b2, W3, b3, lg, lb)


def kernel(track_uri_can, artist_uri_can, album_uri_can, track_name_can,
           artist_name_can, album_name_can, artist_genres_can, track_key_can,
           track_mode_can, time_signature_can, duration_ms_can, track_pop_can,
           artist_pop_can, artists_followers_can, track_danceability_can,
           track_energy_can, track_loudness_can, track_speechiness_can,
           track_acousticness_can, track_instrumentalness_can,
           track_liveness_can, track_valence_can, track_tempo_can,
           E_track_uri, E_artist_uri, E_album_uri, E_track_name,
           E_artist_name, E_album_name, E_genres, E_duration, E_track_pop,
           E_artist_pop, E_followers, E_dance, E_energy, E_key, E_loud,
           E_mode, E_speech, E_acoustic, E_instr, E_live, E_valence, E_tempo,
           E_timesig, cross_U, cross_V, cross_b, W1, b1, W2, b2, W3, b3,
           ln_g, ln_b):
    i32 = jnp.int32
    f32 = jnp.float32

    idx1 = track_uri_can.astype(i32)
    idx2 = artist_uri_can.astype(i32)
    idx3 = album_uri_can.astype(i32)
    idxT4 = track_name_can.astype(i32).T
    idxT5 = artist_name_can.astype(i32).T
    idxT6 = album_name_can.astype(i32).T
    idxT7 = artist_genres_can.astype(i32).T

    def quad_view(tab):
        x = _quad_rows(tab.shape[0])
        return tab.reshape(-1)[:x * 128].reshape(x, 128)

    scd = _sc_direct(idx1, idx2, idx3, quad_view(E_track_uri),
                     quad_view(E_artist_uri), quad_view(E_album_uri))
    scp = _sc_pooled(idxT4, idxT5, idxT6, idxT7,
                     E_track_name, E_artist_name, E_album_name, E_genres)

    vals = [duration_ms_can, track_pop_can, artist_pop_can,
            artists_followers_can, track_danceability_can, track_energy_can,
            track_loudness_can, track_speechiness_can, track_acousticness_can,
            track_instrumentalness_can, track_liveness_can, track_valence_can,
            track_tempo_can]
    maxvs = [20744575.0, 100.0, 100.0, 94437255.0, 1.0, 1.0, 5.0, 1.0, 1.0,
             1.0, 1.0, 1.0, 250.0]
    vals_pack = jnp.pad(jnp.stack([v.astype(f32) for v in vals], axis=1),
                        ((0, 0), (0, 3)))

    idx_pack = jnp.pad(jnp.stack([track_key_can.astype(i32),
                                  track_mode_can.astype(i32),
                                  time_signature_can.astype(i32),
                                  idx1, idx2, idx3], axis=1),
                       ((0, 0), (0, 2)))

    gpad = jnp.pad(artist_genres_can.astype(i32), ((0, 0), (0, 32 - L)))

    bins_np = np.full((16, 32), np.inf, dtype=np.float32)
    for f, mv in enumerate(maxvs):
        bins_np[f, :20] = np.linspace(0.0, mv, 20, dtype=np.float32)
    bins_pack = jnp.asarray(bins_np)

    def pad32(tab):
        return jnp.pad(tab.astype(f32), ((0, 32 - tab.shape[0]), (0, 0)))

    dtabs = jnp.stack([pad32(t) for t in
                       [E_duration, E_track_pop, E_artist_pop, E_followers,
                        E_dance, E_energy, E_loud, E_speech, E_acoustic,
                        E_instr, E_live, E_valence, E_tempo]])
    ptabs = jnp.stack([pad32(E_key), pad32(E_mode), pad32(E_timesig)])
    g0 = E_genres[0:1].astype(f32)
    aux_start = 4 * _quad_rows(N_TRACK)
    aux = pad32(E_track_uri[aux_start:])

    return _tower(scd, scp, vals_pack, idx_pack, gpad, bins_pack, dtabs,
                  ptabs, g0, aux, cross_U.astype(f32), cross_V.astype(f32),
                  cross_b.astype(f32).reshape(1, -1), W1.astype(f32),
                  b1.astype(f32).reshape(1, -1), W2.astype(f32),
                  b2.astype(f32).reshape(1, -1), W3.astype(f32),
                  b3.astype(f32).reshape(1, -1), ln_g.astype(f32).reshape(1, -1),
                  ln_b.astype(f32).reshape(1, -1))
